# baseline (speedup 1.0000x reference)
import functools

import numpy as np
import jax
import jax.numpy as jnp
from jax import lax
from jax.experimental import pallas as pl
from jax.experimental.pallas import tpu as pltpu
from jax.experimental.pallas import tpu_sc as plsc

LEVELS = 16
HASH_SIZE = 1 << 20
HASH_MASK = HASH_SIZE - 1
BASE = 16.0
FINEST = 2048.0
RATIO = float(np.exp((np.log(FINEST) - np.log(BASE)) / (LEVELS - 1)))
RES = np.array([int(np.floor(BASE * (RATIO ** l))) for l in range(LEVELS)],
               dtype=np.float32)
P1 = np.int32(np.uint32(2654435761).astype(np.int32))
P2 = np.int32(np.uint32(805459861).astype(np.int32))
N_POINTS = 262144
UNITS = 64
TAB_ROWS = LEVELS * HASH_SIZE // 4

NW = 32
PW = N_POINTS // NW
C = 32
NCHUNK = PW // C
NG = C // 16


def _sc_body(pos_hbm, geo_hbm, col_hbm, res_hbm, enc_hbm, mask_hbm,
             pos_v, idx_buf, rem_buf, grows_v, crows_v, w_buf, enc_buf,
             mask_buf, res_v, sem):
    wid = lax.axis_index("s") * 2 + lax.axis_index("c")
    gtab = geo_hbm
    ctab = col_hbm
    pltpu.sync_copy(res_hbm, res_v)
    iota = lax.iota(jnp.int32, 16)

    def chunk_body(chunk, _):
        pbase = wid * PW + chunk * C
        pltpu.sync_copy(pos_hbm.at[pl.ds(pbase * 3, C * 3)], pos_v)

        def p1_group(g, _):
            i3 = iota * 3 + g * 48
            x = plsc.load_gather(pos_v, [i3])
            y = plsc.load_gather(pos_v, [i3 + 1])
            z = plsc.load_gather(pos_v, [i3 + 2])
            xc = jnp.minimum(jnp.maximum(x, -1.0), 1.0)
            yc = jnp.minimum(jnp.maximum(y, -1.0), 1.0)
            zc = jnp.minimum(jnp.maximum(z, -1.0), 1.0)
            inb = jnp.logical_and(jnp.logical_and(x == xc, y == yc), z == zc)
            mask_buf[pl.ds(g * 16, 16)] = jnp.where(inb, 1.0, 0.0).astype(jnp.float32)
            lx = (xc + 1.0) * 0.5
            ly = (yc + 1.0) * 0.5
            lz = (zc + 1.0) * 0.5
            ivec = iota + g * 16

            def p1_level(l, _):
                res = plsc.load_gather(res_v, [jnp.full((16,), l, jnp.int32)])
                px = lx * res
                py = ly * res
                pz = lz * res
                ix = px.astype(jnp.int32)
                iy = py.astype(jnp.int32)
                iz = pz.astype(jnp.int32)
                fx = px - ix.astype(jnp.float32)
                fy = py - iy.astype(jnp.float32)
                fz = pz - iz.astype(jnp.float32)
                hx = (ix, ix + 1)
                hy = (iy * P1, iy * P1 + P1)
                hz = (iz * P2, iz * P2 + P2)
                wx1, wx0 = fx, 1.0 - fx
                wy1, wy0 = fy, 1.0 - fy
                wz = (1.0 - fz, fz)
                wxy = (wx0 * wy0, wx1 * wy0, wx0 * wy1, wx1 * wy1)
                lbase = l * HASH_SIZE
                for c in range(8):
                    bx, by, bz = c & 1, (c >> 1) & 1, (c >> 2) & 1
                    h = (hx[bx] ^ hy[by] ^ hz[bz]) & HASH_MASK
                    jv = jnp.full((16,), l * 8 + c, jnp.int32)
                    plsc.store_scatter(idx_buf, [ivec, jv],
                                       lax.shift_right_logical(h + lbase, 2))
                    plsc.store_scatter(rem_buf, [ivec, jv],
                                       lax.shift_left(h & 3, 1))
                    w_off = ((g * 16 + l) * 8 + c) * 16
                    w_buf[pl.ds(w_off, 16)] = wxy[c & 3] * wz[bz]
                return 0

            lax.fori_loop(0, LEVELS, p1_level, 0)
            return 0

        lax.fori_loop(0, NG, p1_group, 0)

        def fire(j, _):
            pltpu.async_copy(gtab.at[idx_buf.at[j]], grows_v.at[j], sem)
            pltpu.async_copy(ctab.at[idx_buf.at[j]], crows_v.at[j], sem)
            return 0

        def drain(j, _):
            pltpu.make_async_copy(gtab.at[idx_buf.at[j]], grows_v.at[j],
                                  sem).wait()
            pltpu.make_async_copy(ctab.at[idx_buf.at[j]], crows_v.at[j],
                                  sem).wait()
            return 0

        lax.fori_loop(0, C, fire, 0)
        lax.fori_loop(0, C, drain, 0)

        def p2_group(g, _):
            ivec = iota + g * 16
            e64 = iota * UNITS + g * (16 * UNITS)

            def p2_level(l, _):
                acc = [jnp.zeros((16,), jnp.float32) for _ in range(4)]
                for c in range(8):
                    w_off = ((g * 16 + l) * 8 + c) * 16
                    w = w_buf[pl.ds(w_off, 16)]
                    jv = jnp.full((16,), l * 8 + c, jnp.int32)
                    rem2 = plsc.load_gather(rem_buf, [ivec, jv])
                    acc[0] = acc[0] + w * plsc.load_gather(grows_v, [ivec, jv, rem2])
                    acc[1] = acc[1] + w * plsc.load_gather(grows_v, [ivec, jv, rem2 + 1])
                    acc[2] = acc[2] + w * plsc.load_gather(crows_v, [ivec, jv, rem2])
                    acc[3] = acc[3] + w * plsc.load_gather(crows_v, [ivec, jv, rem2 + 1])
                for f in range(4):
                    plsc.store_scatter(enc_buf, [e64 + (l * 4 + f)], acc[f])
                return 0

            lax.fori_loop(0, LEVELS, p2_level, 0)
            return 0

        lax.fori_loop(0, NG, p2_group, 0)

        pltpu.sync_copy(enc_buf, enc_hbm.at[pl.ds(pbase * UNITS, C * UNITS)])
        pltpu.sync_copy(mask_buf, mask_hbm.at[pl.ds(pbase, C)])
        return 0

    lax.fori_loop(0, NCHUNK, chunk_body, 0)


def _make_sc_encoder():
    mesh = plsc.VectorSubcoreMesh(core_axis_name="c", subcore_axis_name="s")
    return pl.kernel(
        _sc_body,
        mesh=mesh,
        compiler_params=pltpu.CompilerParams(needs_layout_passes=False,
                                             use_tc_tiling_on_sc=False),
        out_type=[
            jax.ShapeDtypeStruct((N_POINTS * UNITS,), jnp.float32),
            jax.ShapeDtypeStruct((N_POINTS,), jnp.float32),
        ],
        scratch_types=[
            pltpu.VMEM((C * 3,), jnp.float32),
            pltpu.VMEM((C, LEVELS * 8), jnp.int32),
            pltpu.VMEM((C, LEVELS * 8), jnp.int32),
            pltpu.VMEM((C, LEVELS * 8, 8), jnp.float32),
            pltpu.VMEM((C, LEVELS * 8, 8), jnp.float32),
            pltpu.VMEM((C * LEVELS * 8,), jnp.float32),
            pltpu.VMEM((C * UNITS,), jnp.float32),
            pltpu.VMEM((C,), jnp.float32),
            pltpu.VMEM((LEVELS,), jnp.float32),
            pltpu.SemaphoreType.DMA,
        ],
    )


def _mlp_body(enc_ref, w1_ref, w2_ref, geo_ref, col_ref):
    h = jnp.maximum(jnp.dot(enc_ref[...], w1_ref[...],
                            preferred_element_type=jnp.float32), 0.0)
    o = jnp.maximum(jnp.dot(h, w2_ref[...],
                            preferred_element_type=jnp.float32), 0.0)
    geo_ref[...] = o[:, :UNITS]
    col_ref[...] = o[:, UNITS:]


_BN = 1024


def _run_mlp(enc, w1b, w2b):
    grid = (N_POINTS // _BN,)
    return pl.pallas_call(
        _mlp_body,
        grid=grid,
        in_specs=[
            pl.BlockSpec((_BN, UNITS), lambda i: (i, 0)),
            pl.BlockSpec((UNITS, 2 * UNITS), lambda i: (0, 0)),
            pl.BlockSpec((2 * UNITS, 2 * UNITS), lambda i: (0, 0)),
        ],
        out_specs=[
            pl.BlockSpec((_BN, UNITS), lambda i: (i, 0)),
            pl.BlockSpec((_BN, UNITS), lambda i: (i, 0)),
        ],
        out_shape=[
            jax.ShapeDtypeStruct((N_POINTS, UNITS), jnp.float32),
            jax.ShapeDtypeStruct((N_POINTS, UNITS), jnp.float32),
        ],
    )(enc, w1b, w2b)


def kernel(pos_xyz, geo_table, geo_W1, geo_W2, color_table, color_W1, color_W2):
    pos_flat = pos_xyz.reshape(-1)
    geo_flat = geo_table.reshape(TAB_ROWS, 8)
    col_flat = color_table.reshape(TAB_ROWS, 8)
    res_arr = jnp.asarray(RES)

    enc_flat, mask = _make_sc_encoder()(pos_flat, geo_flat, col_flat, res_arr)
    enc = enc_flat.reshape(N_POINTS, UNITS)

    g1 = geo_W1.reshape(LEVELS, 2, UNITS)
    c1 = color_W1.reshape(LEVELS, 2, UNITS)
    w1b = jnp.zeros((LEVELS, 4, 2 * UNITS), jnp.float32)
    w1b = w1b.at[:, 0:2, :UNITS].set(g1).at[:, 2:4, UNITS:].set(c1)
    w1b = w1b.reshape(4 * LEVELS, 2 * UNITS)
    w2b = jnp.zeros((2 * UNITS, 2 * UNITS), jnp.float32)
    w2b = w2b.at[:UNITS, :UNITS].set(geo_W2).at[UNITS:, UNITS:].set(color_W2)

    geo_out, col_out = _run_mlp(enc, w1b, w2b)
    return (geo_out, col_out, mask)


# speedup vs baseline: 4.8433x; 4.8433x over previous
import functools

import numpy as np
import jax
import jax.numpy as jnp
from jax import lax
from jax.experimental import pallas as pl
from jax.experimental.pallas import tpu as pltpu
from jax.experimental.pallas import tpu_sc as plsc

LEVELS = 16
HASH_SIZE = 1 << 20
HASH_MASK = HASH_SIZE - 1
BASE = 16.0
FINEST = 2048.0
RATIO = float(np.exp((np.log(FINEST) - np.log(BASE)) / (LEVELS - 1)))
RES = np.array([int(np.floor(BASE * (RATIO ** l))) for l in range(LEVELS)],
               dtype=np.float32)
P1 = np.int32(np.uint32(2654435761).astype(np.int32))
P2 = np.int32(np.uint32(805459861).astype(np.int32))
N_POINTS = 262144
UNITS = 64
PLANE_ROWS = HASH_SIZE // 8
LEVEL_ROWS = 2 * PLANE_ROWS
TAB_ROWS = LEVELS * LEVEL_ROWS

NW = 32
PW = N_POINTS // NW
C = 16
NCHUNK = PW // C


def _sc_body(pos_hbm, geo_hbm, col_hbm, res_hbm, enc_hbm, mask_hbm,
             pos_v, idx0_buf, idx1_buf, rem_buf, g0_v, g1_v, c0_v, c1_v,
             w_buf, enc_buf, mask_buf, res_v, sem):
    wid = lax.axis_index("s") * 2 + lax.axis_index("c")
    pltpu.sync_copy(res_hbm, res_v)
    iota = lax.iota(jnp.int32, 16)

    def chunk_body(chunk, _):
        pbase = wid * PW + chunk * C
        pltpu.sync_copy(pos_hbm.at[pl.ds(pbase, C)], pos_v.at[pl.ds(0, C)])
        pltpu.sync_copy(pos_hbm.at[pl.ds(N_POINTS + pbase, C)],
                        pos_v.at[pl.ds(C, C)])
        pltpu.sync_copy(pos_hbm.at[pl.ds(2 * N_POINTS + pbase, C)],
                        pos_v.at[pl.ds(2 * C, C)])

        x = pos_v[pl.ds(0, 16)]
        y = pos_v[pl.ds(C, 16)]
        z = pos_v[pl.ds(2 * C, 16)]
        xc = jnp.minimum(jnp.maximum(x, -1.0), 1.0)
        yc = jnp.minimum(jnp.maximum(y, -1.0), 1.0)
        zc = jnp.minimum(jnp.maximum(z, -1.0), 1.0)
        inb = jnp.logical_and(jnp.logical_and(x == xc, y == yc), z == zc)
        mask_buf[pl.ds(0, 16)] = jnp.where(inb, 1.0, 0.0).astype(jnp.float32)
        lx = (xc + 1.0) * 0.5
        ly = (yc + 1.0) * 0.5
        lz = (zc + 1.0) * 0.5

        def p1_level(l, _):
            res = plsc.load_gather(res_v, [jnp.full((16,), l, jnp.int32)])
            px = lx * res
            py = ly * res
            pz = lz * res
            ix = px.astype(jnp.int32)
            iy = py.astype(jnp.int32)
            iz = pz.astype(jnp.int32)
            fx = px - ix.astype(jnp.float32)
            fy = py - iy.astype(jnp.float32)
            fz = pz - iz.astype(jnp.float32)
            hx = (ix, ix + 1)
            hy = (iy * P1, iy * P1 + P1)
            hz = (iz * P2, iz * P2 + P2)
            wx1, wx0 = fx, 1.0 - fx
            wy1, wy0 = fy, 1.0 - fy
            wz = (1.0 - fz, fz)
            wxy = (wx0 * wy0, wx1 * wy0, wx0 * wy1, wx1 * wy1)
            lbase = l * LEVEL_ROWS
            for c in range(8):
                bx, by, bz = c & 1, (c >> 1) & 1, (c >> 2) & 1
                h = (hx[bx] ^ hy[by] ^ hz[bz]) & HASH_MASK
                jv = jnp.full((16,), l * 8 + c, jnp.int32)
                t = lax.shift_right_logical(h, 3)
                r0 = (lbase + lax.shift_left(lax.shift_right_logical(t, 4), 5)
                      + (t & 15))
                plsc.store_scatter(idx0_buf, [iota, jv], r0)
                plsc.store_scatter(idx1_buf, [iota, jv], r0 + 16)
                plsc.store_scatter(rem_buf, [iota, jv], h & 7)
                w_off = (l * 8 + c) * 16
                w_buf[pl.ds(w_off, 16)] = wxy[c & 3] * wz[bz]
            return 0

        lax.fori_loop(0, LEVELS, p1_level, 0)

        def fire(j, _):
            pltpu.async_copy(geo_hbm.at[idx0_buf.at[j]], g0_v.at[j], sem)
            pltpu.async_copy(geo_hbm.at[idx1_buf.at[j]], g1_v.at[j], sem)
            pltpu.async_copy(col_hbm.at[idx0_buf.at[j]], c0_v.at[j], sem)
            pltpu.async_copy(col_hbm.at[idx1_buf.at[j]], c1_v.at[j], sem)
            return 0

        def drain(j, _):
            pltpu.make_async_copy(geo_hbm.at[idx0_buf.at[j]], g0_v.at[j],
                                  sem).wait()
            pltpu.make_async_copy(geo_hbm.at[idx1_buf.at[j]], g1_v.at[j],
                                  sem).wait()
            pltpu.make_async_copy(col_hbm.at[idx0_buf.at[j]], c0_v.at[j],
                                  sem).wait()
            pltpu.make_async_copy(col_hbm.at[idx1_buf.at[j]], c1_v.at[j],
                                  sem).wait()
            return 0

        lax.fori_loop(0, C, fire, 0)
        lax.fori_loop(0, C, drain, 0)

        e64 = iota * UNITS

        def p2_level(l, _):
            acc = [jnp.zeros((16,), jnp.float32) for _ in range(4)]
            for c in range(8):
                w = w_buf[pl.ds((l * 8 + c) * 16, 16)]
                jv = jnp.full((16,), l * 8 + c, jnp.int32)
                rem = plsc.load_gather(rem_buf, [iota, jv])
                acc[0] = acc[0] + w * plsc.load_gather(g0_v, [iota, jv, rem])
                acc[1] = acc[1] + w * plsc.load_gather(g1_v, [iota, jv, rem])
                acc[2] = acc[2] + w * plsc.load_gather(c0_v, [iota, jv, rem])
                acc[3] = acc[3] + w * plsc.load_gather(c1_v, [iota, jv, rem])
            for f in range(4):
                plsc.store_scatter(enc_buf, [e64 + (l * 4 + f)], acc[f])
            return 0

        lax.fori_loop(0, LEVELS, p2_level, 0)

        pltpu.sync_copy(enc_buf, enc_hbm.at[pl.ds(pbase * UNITS, C * UNITS)])
        pltpu.sync_copy(mask_buf, mask_hbm.at[pl.ds(pbase, C)])
        return 0

    lax.fori_loop(0, NCHUNK, chunk_body, 0)


def _make_sc_encoder():
    mesh = plsc.VectorSubcoreMesh(core_axis_name="c", subcore_axis_name="s")
    return pl.kernel(
        _sc_body,
        mesh=mesh,
        compiler_params=pltpu.CompilerParams(needs_layout_passes=False,
                                             use_tc_tiling_on_sc=False),
        out_type=[
            jax.ShapeDtypeStruct((N_POINTS * UNITS,), jnp.float32),
            jax.ShapeDtypeStruct((N_POINTS,), jnp.float32),
        ],
        scratch_types=[
            pltpu.VMEM((C * 3,), jnp.float32),
            pltpu.VMEM((C, LEVELS * 8), jnp.int32),
            pltpu.VMEM((C, LEVELS * 8), jnp.int32),
            pltpu.VMEM((C, LEVELS * 8), jnp.int32),
            pltpu.VMEM((C, LEVELS * 8, 8), jnp.float32),
            pltpu.VMEM((C, LEVELS * 8, 8), jnp.float32),
            pltpu.VMEM((C, LEVELS * 8, 8), jnp.float32),
            pltpu.VMEM((C, LEVELS * 8, 8), jnp.float32),
            pltpu.VMEM((C * LEVELS * 8,), jnp.float32),
            pltpu.VMEM((C * UNITS,), jnp.float32),
            pltpu.VMEM((C,), jnp.float32),
            pltpu.VMEM((LEVELS,), jnp.float32),
            pltpu.SemaphoreType.DMA,
        ],
    )


def _mlp_body(enc_ref, w1_ref, w2_ref, geo_ref, col_ref):
    h = jnp.maximum(jnp.dot(enc_ref[...], w1_ref[...],
                            preferred_element_type=jnp.float32), 0.0)
    o = jnp.maximum(jnp.dot(h, w2_ref[...],
                            preferred_element_type=jnp.float32), 0.0)
    geo_ref[...] = o[:, :UNITS]
    col_ref[...] = o[:, UNITS:]


_BN = 1024


def _run_mlp(enc, w1b, w2b):
    grid = (N_POINTS // _BN,)
    return pl.pallas_call(
        _mlp_body,
        grid=grid,
        in_specs=[
            pl.BlockSpec((_BN, UNITS), lambda i: (i, 0)),
            pl.BlockSpec((UNITS, 2 * UNITS), lambda i: (0, 0)),
            pl.BlockSpec((2 * UNITS, 2 * UNITS), lambda i: (0, 0)),
        ],
        out_specs=[
            pl.BlockSpec((_BN, UNITS), lambda i: (i, 0)),
            pl.BlockSpec((_BN, UNITS), lambda i: (i, 0)),
        ],
        out_shape=[
            jax.ShapeDtypeStruct((N_POINTS, UNITS), jnp.float32),
            jax.ShapeDtypeStruct((N_POINTS, UNITS), jnp.float32),
        ],
    )(enc, w1b, w2b)


def kernel(pos_xyz, geo_table, geo_W1, geo_W2, color_table, color_W1, color_W2):
    pos_planes = pos_xyz.transpose(1, 0).reshape(-1)
    gtab = geo_table.reshape(LEVELS, 8192, 128, 2).transpose(
        0, 1, 3, 2).reshape(TAB_ROWS, 8)
    ctab = color_table.reshape(LEVELS, 8192, 128, 2).transpose(
        0, 1, 3, 2).reshape(TAB_ROWS, 8)
    res_arr = jnp.asarray(RES)

    enc_flat, mask = _make_sc_encoder()(pos_planes, gtab, ctab, res_arr)
    enc = enc_flat.reshape(N_POINTS, UNITS)

    g1 = geo_W1.reshape(LEVELS, 2, UNITS)
    c1 = color_W1.reshape(LEVELS, 2, UNITS)
    w1b = jnp.zeros((LEVELS, 4, 2 * UNITS), jnp.float32)
    w1b = w1b.at[:, 0:2, :UNITS].set(g1).at[:, 2:4, UNITS:].set(c1)
    w1b = w1b.reshape(4 * LEVELS, 2 * UNITS)
    w2b = jnp.zeros((2 * UNITS, 2 * UNITS), jnp.float32)
    w2b = w2b.at[:UNITS, :UNITS].set(geo_W2).at[UNITS:, UNITS:].set(color_W2)

    geo_out, col_out = _run_mlp(enc, w1b, w2b)
    return (geo_out, col_out, mask)


# speedup vs baseline: 6.1474x; 1.2693x over previous
import functools

import numpy as np
import jax
import jax.numpy as jnp
from jax import lax
from jax.experimental import pallas as pl
from jax.experimental.pallas import tpu as pltpu
from jax.experimental.pallas import tpu_sc as plsc

LEVELS = 16
HASH_SIZE = 1 << 20
HASH_MASK = HASH_SIZE - 1
BASE = 16.0
FINEST = 2048.0
RATIO = float(np.exp((np.log(FINEST) - np.log(BASE)) / (LEVELS - 1)))
RES = np.array([int(np.floor(BASE * (RATIO ** l))) for l in range(LEVELS)],
               dtype=np.float32)
P1 = np.int32(np.uint32(2654435761).astype(np.int32))
P2 = np.int32(np.uint32(805459861).astype(np.int32))
N_POINTS = 262144
UNITS = 64
PLANE_ROWS = HASH_SIZE // 8
LEVEL_ROWS = 2 * PLANE_ROWS
TAB_ROWS = LEVELS * LEVEL_ROWS

NW = 32
PW = N_POINTS // NW
C = 16
NCHUNK = PW // C

S0 = 18
L0_CELLS = S0 * S0 * S0
L0_BLOCKS = (L0_CELLS + 127) // 128
NL = LEVELS - 1


def _sc_body(pos_hbm, geo_hbm, col_hbm, res_hbm, enc_hbm, mask_hbm,
             pos_v, idx0_buf, idx1_buf, rem_buf, g0_v, g1_v, c0_v, c1_v,
             w_buf, enc_buf, mask_buf, res_v, grid_v,
             sidx0, sidx1, srem, sg0, sg1, sc0, sc1, sem):
    wid = lax.axis_index("s") * 2 + lax.axis_index("c")
    pltpu.sync_copy(res_hbm, res_v)
    iota = lax.iota(jnp.int32, 16)

    def stage_blk(blk, _):
        def stage_grp(g, _):
            cid = jnp.minimum(iota + g * 16 + blk * 128, L0_CELLS - 1)
            z = cid // (S0 * S0)
            r = cid - z * (S0 * S0)
            y = r // S0
            x = r - y * S0
            h = (x ^ (y * P1) ^ (z * P2)) & HASH_MASK
            t = lax.shift_right_logical(h, 3)
            r0 = (lax.shift_left(lax.shift_right_logical(t, 4), 5)
                  + (t & 15))
            lv = iota + g * 16
            plsc.store_scatter(sidx0, [lv], r0)
            plsc.store_scatter(sidx1, [lv], r0 + 16)
            plsc.store_scatter(srem, [lv], h & 7)
            return 0

        lax.fori_loop(0, 8, stage_grp, 0)
        pltpu.async_copy(geo_hbm.at[sidx0], sg0, sem)
        pltpu.async_copy(geo_hbm.at[sidx1], sg1, sem)
        pltpu.async_copy(col_hbm.at[sidx0], sc0, sem)
        pltpu.async_copy(col_hbm.at[sidx1], sc1, sem)
        pltpu.make_async_copy(geo_hbm.at[sidx0], sg0, sem).wait()
        pltpu.make_async_copy(geo_hbm.at[sidx1], sg1, sem).wait()
        pltpu.make_async_copy(col_hbm.at[sidx0], sc0, sem).wait()
        pltpu.make_async_copy(col_hbm.at[sidx1], sc1, sem).wait()

        def asm_grp(g, _):
            lv = iota + g * 16
            cid = jnp.minimum(lv + blk * 128, L0_CELLS - 1)
            rem = plsc.load_gather(srem, [lv])
            b4 = cid * 4
            plsc.store_scatter(grid_v, [b4],
                               plsc.load_gather(sg0, [lv, rem]))
            plsc.store_scatter(grid_v, [b4 + 1],
                               plsc.load_gather(sg1, [lv, rem]))
            plsc.store_scatter(grid_v, [b4 + 2],
                               plsc.load_gather(sc0, [lv, rem]))
            plsc.store_scatter(grid_v, [b4 + 3],
                               plsc.load_gather(sc1, [lv, rem]))
            return 0

        lax.fori_loop(0, 8, asm_grp, 0)
        return 0

    lax.fori_loop(0, L0_BLOCKS, stage_blk, 0)

    def chunk_body(chunk, _):
        pbase = wid * PW + chunk * C
        pltpu.sync_copy(pos_hbm.at[pl.ds(pbase, C)], pos_v.at[pl.ds(0, C)])
        pltpu.sync_copy(pos_hbm.at[pl.ds(N_POINTS + pbase, C)],
                        pos_v.at[pl.ds(C, C)])
        pltpu.sync_copy(pos_hbm.at[pl.ds(2 * N_POINTS + pbase, C)],
                        pos_v.at[pl.ds(2 * C, C)])

        x = pos_v[pl.ds(0, 16)]
        y = pos_v[pl.ds(C, 16)]
        z = pos_v[pl.ds(2 * C, 16)]
        xc = jnp.minimum(jnp.maximum(x, -1.0), 1.0)
        yc = jnp.minimum(jnp.maximum(y, -1.0), 1.0)
        zc = jnp.minimum(jnp.maximum(z, -1.0), 1.0)
        inb = jnp.logical_and(jnp.logical_and(x == xc, y == yc), z == zc)
        mask_buf[pl.ds(0, 16)] = jnp.where(inb, 1.0, 0.0).astype(jnp.float32)
        lx = (xc + 1.0) * 0.5
        ly = (yc + 1.0) * 0.5
        lz = (zc + 1.0) * 0.5
        e64 = iota * UNITS

        px = lx * 16.0
        py = ly * 16.0
        pz = lz * 16.0
        ix = px.astype(jnp.int32)
        iy = py.astype(jnp.int32)
        iz = pz.astype(jnp.int32)
        fx = px - ix.astype(jnp.float32)
        fy = py - iy.astype(jnp.float32)
        fz = pz - iz.astype(jnp.float32)
        cx = (ix, ix + 1)
        cy = (iy * S0, iy * S0 + S0)
        cz = (iz * (S0 * S0), iz * (S0 * S0) + S0 * S0)
        wx1, wx0 = fx, 1.0 - fx
        wy1, wy0 = fy, 1.0 - fy
        wz0l = (1.0 - fz, fz)
        wxy0 = (wx0 * wy0, wx1 * wy0, wx0 * wy1, wx1 * wy1)
        acc0 = [jnp.zeros((16,), jnp.float32) for _ in range(4)]
        for c in range(8):
            bx, by, bz = c & 1, (c >> 1) & 1, (c >> 2) & 1
            b4 = (cx[bx] + cy[by] + cz[bz]) * 4
            w = wxy0[c & 3] * wz0l[bz]
            for f in range(4):
                acc0[f] = acc0[f] + w * plsc.load_gather(grid_v, [b4 + f])
        for f in range(4):
            plsc.store_scatter(enc_buf, [e64 + f], acc0[f])

        def p1_level(l, _):
            res = plsc.load_gather(res_v, [jnp.full((16,), l, jnp.int32)])
            px = lx * res
            py = ly * res
            pz = lz * res
            ix = px.astype(jnp.int32)
            iy = py.astype(jnp.int32)
            iz = pz.astype(jnp.int32)
            fx = px - ix.astype(jnp.float32)
            fy = py - iy.astype(jnp.float32)
            fz = pz - iz.astype(jnp.float32)
            hx = (ix, ix + 1)
            hy = (iy * P1, iy * P1 + P1)
            hz = (iz * P2, iz * P2 + P2)
            wx1, wx0 = fx, 1.0 - fx
            wy1, wy0 = fy, 1.0 - fy
            wz = (1.0 - fz, fz)
            wxy = (wx0 * wy0, wx1 * wy0, wx0 * wy1, wx1 * wy1)
            lbase = l * LEVEL_ROWS
            for c in range(8):
                bx, by, bz = c & 1, (c >> 1) & 1, (c >> 2) & 1
                h = (hx[bx] ^ hy[by] ^ hz[bz]) & HASH_MASK
                jv = jnp.full((16,), l * 8 + c - 8, jnp.int32)
                t = lax.shift_right_logical(h, 3)
                r0 = (lbase + lax.shift_left(lax.shift_right_logical(t, 4), 5)
                      + (t & 15))
                plsc.store_scatter(idx0_buf, [iota, jv], r0)
                plsc.store_scatter(idx1_buf, [iota, jv], r0 + 16)
                plsc.store_scatter(rem_buf, [iota, jv], h & 7)
                w_off = (l * 8 + c - 8) * 16
                w_buf[pl.ds(w_off, 16)] = wxy[c & 3] * wz[bz]
            return 0

        lax.fori_loop(1, LEVELS, p1_level, 0)

        def fire(j, _):
            pltpu.async_copy(geo_hbm.at[idx0_buf.at[j]], g0_v.at[j], sem)
            pltpu.async_copy(geo_hbm.at[idx1_buf.at[j]], g1_v.at[j], sem)
            pltpu.async_copy(col_hbm.at[idx0_buf.at[j]], c0_v.at[j], sem)
            pltpu.async_copy(col_hbm.at[idx1_buf.at[j]], c1_v.at[j], sem)
            return 0

        def drain(j, _):
            pltpu.make_async_copy(geo_hbm.at[idx0_buf.at[j]], g0_v.at[j],
                                  sem).wait()
            pltpu.make_async_copy(geo_hbm.at[idx1_buf.at[j]], g1_v.at[j],
                                  sem).wait()
            pltpu.make_async_copy(col_hbm.at[idx0_buf.at[j]], c0_v.at[j],
                                  sem).wait()
            pltpu.make_async_copy(col_hbm.at[idx1_buf.at[j]], c1_v.at[j],
                                  sem).wait()
            return 0

        lax.fori_loop(0, C, fire, 0)
        lax.fori_loop(0, C, drain, 0)

        def p2_level(l, _):
            acc = [jnp.zeros((16,), jnp.float32) for _ in range(4)]
            for c in range(8):
                w = w_buf[pl.ds((l * 8 + c - 8) * 16, 16)]
                jv = jnp.full((16,), l * 8 + c - 8, jnp.int32)
                rem = plsc.load_gather(rem_buf, [iota, jv])
                acc[0] = acc[0] + w * plsc.load_gather(g0_v, [iota, jv, rem])
                acc[1] = acc[1] + w * plsc.load_gather(g1_v, [iota, jv, rem])
                acc[2] = acc[2] + w * plsc.load_gather(c0_v, [iota, jv, rem])
                acc[3] = acc[3] + w * plsc.load_gather(c1_v, [iota, jv, rem])
            for f in range(4):
                plsc.store_scatter(enc_buf, [e64 + (l * 4 + f)], acc[f])
            return 0

        lax.fori_loop(1, LEVELS, p2_level, 0)

        pltpu.sync_copy(enc_buf, enc_hbm.at[pl.ds(pbase * UNITS, C * UNITS)])
        pltpu.sync_copy(mask_buf, mask_hbm.at[pl.ds(pbase, C)])
        return 0

    lax.fori_loop(0, NCHUNK, chunk_body, 0)


def _make_sc_encoder():
    mesh = plsc.VectorSubcoreMesh(core_axis_name="c", subcore_axis_name="s")
    return pl.kernel(
        _sc_body,
        mesh=mesh,
        compiler_params=pltpu.CompilerParams(needs_layout_passes=False,
                                             use_tc_tiling_on_sc=False),
        out_type=[
            jax.ShapeDtypeStruct((N_POINTS * UNITS,), jnp.float32),
            jax.ShapeDtypeStruct((N_POINTS,), jnp.float32),
        ],
        scratch_types=[
            pltpu.VMEM((C * 3,), jnp.float32),
            pltpu.VMEM((C, NL * 8), jnp.int32),
            pltpu.VMEM((C, NL * 8), jnp.int32),
            pltpu.VMEM((C, NL * 8), jnp.int32),
            pltpu.VMEM((C, NL * 8, 8), jnp.float32),
            pltpu.VMEM((C, NL * 8, 8), jnp.float32),
            pltpu.VMEM((C, NL * 8, 8), jnp.float32),
            pltpu.VMEM((C, NL * 8, 8), jnp.float32),
            pltpu.VMEM((C * NL * 8,), jnp.float32),
            pltpu.VMEM((C * UNITS,), jnp.float32),
            pltpu.VMEM((C,), jnp.float32),
            pltpu.VMEM((LEVELS,), jnp.float32),
            pltpu.VMEM((L0_CELLS * 4,), jnp.float32),
            pltpu.VMEM((128,), jnp.int32),
            pltpu.VMEM((128,), jnp.int32),
            pltpu.VMEM((128,), jnp.int32),
            pltpu.VMEM((128, 8), jnp.float32),
            pltpu.VMEM((128, 8), jnp.float32),
            pltpu.VMEM((128, 8), jnp.float32),
            pltpu.VMEM((128, 8), jnp.float32),
            pltpu.SemaphoreType.DMA,
        ],
    )


def _mlp_body(enc_ref, w1_ref, w2_ref, geo_ref, col_ref):
    h = jnp.maximum(jnp.dot(enc_ref[...], w1_ref[...],
                            preferred_element_type=jnp.float32), 0.0)
    o = jnp.maximum(jnp.dot(h, w2_ref[...],
                            preferred_element_type=jnp.float32), 0.0)
    geo_ref[...] = o[:, :UNITS]
    col_ref[...] = o[:, UNITS:]


_BN = 1024


def _run_mlp(enc, w1b, w2b):
    grid = (N_POINTS // _BN,)
    return pl.pallas_call(
        _mlp_body,
        grid=grid,
        in_specs=[
            pl.BlockSpec((_BN, UNITS), lambda i: (i, 0)),
            pl.BlockSpec((UNITS, 2 * UNITS), lambda i: (0, 0)),
            pl.BlockSpec((2 * UNITS, 2 * UNITS), lambda i: (0, 0)),
        ],
        out_specs=[
            pl.BlockSpec((_BN, UNITS), lambda i: (i, 0)),
            pl.BlockSpec((_BN, UNITS), lambda i: (i, 0)),
        ],
        out_shape=[
            jax.ShapeDtypeStruct((N_POINTS, UNITS), jnp.float32),
            jax.ShapeDtypeStruct((N_POINTS, UNITS), jnp.float32),
        ],
    )(enc, w1b, w2b)


def kernel(pos_xyz, geo_table, geo_W1, geo_W2, color_table, color_W1, color_W2):
    pos_planes = pos_xyz.transpose(1, 0).reshape(-1)
    gtab = geo_table.reshape(LEVELS, 8192, 128, 2).transpose(
        0, 1, 3, 2).reshape(TAB_ROWS, 8)
    ctab = color_table.reshape(LEVELS, 8192, 128, 2).transpose(
        0, 1, 3, 2).reshape(TAB_ROWS, 8)
    res_arr = jnp.asarray(RES)

    enc_flat, mask = _make_sc_encoder()(pos_planes, gtab, ctab, res_arr)
    enc = enc_flat.reshape(N_POINTS, UNITS)

    g1 = geo_W1.reshape(LEVELS, 2, UNITS)
    c1 = color_W1.reshape(LEVELS, 2, UNITS)
    w1b = jnp.zeros((LEVELS, 4, 2 * UNITS), jnp.float32)
    w1b = w1b.at[:, 0:2, :UNITS].set(g1).at[:, 2:4, UNITS:].set(c1)
    w1b = w1b.reshape(4 * LEVELS, 2 * UNITS)
    w2b = jnp.zeros((2 * UNITS, 2 * UNITS), jnp.float32)
    w2b = w2b.at[:UNITS, :UNITS].set(geo_W2).at[UNITS:, UNITS:].set(color_W2)

    geo_out, col_out = _run_mlp(enc, w1b, w2b)
    return (geo_out, col_out, mask)
